# baseline (speedup 1.0000x reference)
import jax
import jax.numpy as jnp
from jax import lax
from jax.experimental import pallas as pl
from jax.experimental.pallas import tpu as pltpu
from jax.experimental.pallas import tpu_sc as plsc

N_ROWS = 16384
N_FEAT = 2048
NC = 2
NS = 16
L = 16
NW = NC * NS
ROWS_PER_W = N_ROWS // NW
R = 16
CHUNKS = ROWS_PER_W // R
JV = N_FEAT // L


def _permute_body(x_hbm, idx_hbm, out_hbm, idx_v, in_v, out_v):
    wid = lax.axis_index("s") * NC + lax.axis_index("c")
    base = wid * (ROWS_PER_W * N_FEAT)

    pltpu.sync_copy(idx_hbm, idx_v)

    def chunk_body(c, carry):
        off = base + c * (R * N_FEAT)
        pltpu.sync_copy(x_hbm.at[pl.ds(off, R * N_FEAT)], in_v)

        def j_body(j, carry2):
            iv = idx_v[pl.ds(j * L, L)]
            jl = j * L
            for r in range(R):
                v = plsc.load_gather(in_v, [iv + (r * N_FEAT)])
                out_v[pl.ds(r * N_FEAT + jl, L)] = v
            return carry2

        lax.fori_loop(0, JV, j_body, 0, unroll=2)
        pltpu.sync_copy(out_v, out_hbm.at[pl.ds(off, R * N_FEAT)])
        return carry

    lax.fori_loop(0, CHUNKS, chunk_body, 0)


_permute = pl.kernel(
    _permute_body,
    out_type=jax.ShapeDtypeStruct((N_ROWS * N_FEAT,), jnp.float32),
    mesh=plsc.VectorSubcoreMesh(
        core_axis_name="c", subcore_axis_name="s", num_cores=NC, num_subcores=NS
    ),
    scratch_types=[
        pltpu.VMEM((N_FEAT,), jnp.int32),
        pltpu.VMEM((R * N_FEAT,), jnp.float32),
        pltpu.VMEM((R * N_FEAT,), jnp.float32),
    ],
    compiler_params=pltpu.CompilerParams(needs_layout_passes=False),
)


@jax.jit
def kernel(x, permutation):
    perm = permutation.astype(jnp.int32)
    out_flat = _permute(x.reshape(-1), perm)
    return out_flat.reshape(N_ROWS, N_FEAT)


# speedup vs baseline: 1.1333x; 1.1333x over previous
import jax
import jax.numpy as jnp
from jax import lax
from jax.experimental import pallas as pl
from jax.experimental.pallas import tpu as pltpu
from jax.experimental.pallas import tpu_sc as plsc

N_ROWS = 16384
N_FEAT = 2048
NC = 2
NS = 16
L = 16
NW = NC * NS
ROWS_PER_W = N_ROWS // NW
R = 8
CHUNKS = ROWS_PER_W // R
JV = N_FEAT // L
NBUF = 2


def _permute_body(x_hbm, idx_hbm, out_hbm,
                  idx_v, in0, in1, out0, out1, si0, si1, so0, so1):
    in_bufs = (in0, in1)
    out_bufs = (out0, out1)
    isems = (si0, si1)
    osems = (so0, so1)

    wid = lax.axis_index("s") * NC + lax.axis_index("c")
    base = wid * (ROWS_PER_W * N_FEAT)

    pltpu.sync_copy(idx_hbm, idx_v)

    def x_slice(c):
        return x_hbm.at[pl.ds(base + c * (R * N_FEAT), R * N_FEAT)]

    def o_slice(c):
        return out_hbm.at[pl.ds(base + c * (R * N_FEAT), R * N_FEAT)]

    pltpu.async_copy(x_slice(0), in_bufs[0], isems[0])

    def loop_body(c0, carry):
        for b in range(NBUF):
            c = c0 * NBUF + b

            @pl.when(c + 1 < CHUNKS)
            def _start_next_in():
                pltpu.async_copy(x_slice(c + 1), in_bufs[1 - b], isems[1 - b])

            pltpu.make_async_copy(x_slice(c), in_bufs[b], isems[b]).wait()

            @pl.when(c >= NBUF)
            def _wait_prev_out():
                pltpu.make_async_copy(out_bufs[b], o_slice(c), osems[b]).wait()

            def j_body(j, carry2):
                iv = idx_v[pl.ds(j * L, L)]
                jl = j * L
                for r in range(R):
                    v = plsc.load_gather(in_bufs[b], [iv + (r * N_FEAT)])
                    out_bufs[b][pl.ds(r * N_FEAT + jl, L)] = v
                return carry2

            lax.fori_loop(0, JV, j_body, 0, unroll=4)
            pltpu.async_copy(out_bufs[b], o_slice(c), osems[b])
        return carry

    lax.fori_loop(0, CHUNKS // NBUF, loop_body, 0)

    for b in range(NBUF):
        pltpu.make_async_copy(out_bufs[b], o_slice(CHUNKS - NBUF + b),
                              osems[b]).wait()


_permute = pl.kernel(
    _permute_body,
    out_type=jax.ShapeDtypeStruct((N_ROWS * N_FEAT,), jnp.float32),
    mesh=plsc.VectorSubcoreMesh(
        core_axis_name="c", subcore_axis_name="s", num_cores=NC, num_subcores=NS
    ),
    scratch_types=[
        pltpu.VMEM((N_FEAT,), jnp.int32),
        pltpu.VMEM((R * N_FEAT,), jnp.float32),
        pltpu.VMEM((R * N_FEAT,), jnp.float32),
        pltpu.VMEM((R * N_FEAT,), jnp.float32),
        pltpu.VMEM((R * N_FEAT,), jnp.float32),
        pltpu.SemaphoreType.DMA,
        pltpu.SemaphoreType.DMA,
        pltpu.SemaphoreType.DMA,
        pltpu.SemaphoreType.DMA,
    ],
    compiler_params=pltpu.CompilerParams(needs_layout_passes=False),
)


@jax.jit
def kernel(x, permutation):
    perm = permutation.astype(jnp.int32)
    return _permute(x.reshape(-1), perm).reshape(N_ROWS, N_FEAT)


# speedup vs baseline: 1.8368x; 1.6207x over previous
import jax
import jax.numpy as jnp
from jax import lax
from jax.experimental import pallas as pl
from jax.experimental.pallas import tpu as pltpu
from jax.experimental.pallas import tpu_sc as plsc

N_ROWS = 16384
N_FEAT = 2048
NC = 2
NS = 16
L = 16
NW = NC * NS
ROWS_PER_W = N_ROWS // NW
R = 8
CHUNKS = ROWS_PER_W // R
JV = N_FEAT // L
NBUF = 2


def _permute_body(x_hbm, idx_hbm, out_hbm,
                  idx_v, in0, in1, out0, out1, si0, si1, so0, so1):
    in_bufs = (in0, in1)
    out_bufs = (out0, out1)
    isems = (si0, si1)
    osems = (so0, so1)

    wid = lax.axis_index("s") * NC + lax.axis_index("c")
    base = wid * (ROWS_PER_W * N_FEAT)

    pltpu.sync_copy(idx_hbm, idx_v)

    def x_slice(c):
        return x_hbm.at[pl.ds(base + c * (R * N_FEAT), R * N_FEAT)]

    def o_slice(c):
        return out_hbm.at[pl.ds(base + c * (R * N_FEAT), R * N_FEAT)]

    pltpu.async_copy(x_slice(0), in_bufs[0], isems[0])

    def loop_body(c0, carry):
        for b in range(NBUF):
            c = c0 * NBUF + b

            @pl.when(c + 1 < CHUNKS)
            def _start_next_in():
                pltpu.async_copy(x_slice(c + 1), in_bufs[1 - b], isems[1 - b])

            pltpu.make_async_copy(x_slice(c), in_bufs[b], isems[b]).wait()

            @pl.when(c >= NBUF)
            def _wait_prev_out():
                pltpu.make_async_copy(out_bufs[b], o_slice(c), osems[b]).wait()

            @plsc.parallel_loop(0, JV, 1, unroll=2)
            def _j_body(j):
                iv = idx_v[pl.ds(j * L, L)]
                jl = j * L
                vals = [plsc.load_gather(in_bufs[b], [iv + (r * N_FEAT)])
                        for r in range(R)]
                for r in range(R):
                    out_bufs[b][pl.ds(r * N_FEAT + jl, L)] = vals[r]
            pltpu.async_copy(out_bufs[b], o_slice(c), osems[b])
        return carry

    lax.fori_loop(0, CHUNKS // NBUF, loop_body, 0)

    for b in range(NBUF):
        pltpu.make_async_copy(out_bufs[b], o_slice(CHUNKS - NBUF + b),
                              osems[b]).wait()


_permute = pl.kernel(
    _permute_body,
    out_type=jax.ShapeDtypeStruct((N_ROWS * N_FEAT,), jnp.float32),
    mesh=plsc.VectorSubcoreMesh(
        core_axis_name="c", subcore_axis_name="s", num_cores=NC, num_subcores=NS
    ),
    scratch_types=[
        pltpu.VMEM((N_FEAT,), jnp.int32),
        pltpu.VMEM((R * N_FEAT,), jnp.float32),
        pltpu.VMEM((R * N_FEAT,), jnp.float32),
        pltpu.VMEM((R * N_FEAT,), jnp.float32),
        pltpu.VMEM((R * N_FEAT,), jnp.float32),
        pltpu.SemaphoreType.DMA,
        pltpu.SemaphoreType.DMA,
        pltpu.SemaphoreType.DMA,
        pltpu.SemaphoreType.DMA,
    ],
    compiler_params=pltpu.CompilerParams(needs_layout_passes=False),
)


@jax.jit
def kernel(x, permutation):
    perm = permutation.astype(jnp.int32)
    return _permute(x.reshape(-1), perm).reshape(N_ROWS, N_FEAT)


# speedup vs baseline: 5.2155x; 2.8395x over previous
import jax
import jax.numpy as jnp
from jax import lax
from jax.experimental import pallas as pl
from jax.experimental.pallas import tpu as pltpu
from jax.experimental.pallas import tpu_sc as plsc

N_ROWS = 16384
N_FEAT = 2048
NC = 2
NS = 16
L = 16
NW = NC * NS
ROWS_PER_W = N_ROWS // NW
R = 8
CHUNKS = ROWS_PER_W // R
JV = N_FEAT // L
NBUF = 2


def _permute_body(x_hbm, idx_hbm, out_hbm,
                  idx_v, in0, in1, out0, out1, si0, si1, so0, so1):
    in_bufs = (in0, in1)
    out_bufs = (out0, out1)
    isems = (si0, si1)
    osems = (so0, so1)

    wid = lax.axis_index("s") * NC + lax.axis_index("c")
    rbase = wid * ROWS_PER_W

    pltpu.sync_copy(idx_hbm, idx_v)

    def x_slice(c):
        return x_hbm.at[pl.ds(rbase + c * R, R)]

    def o_slice(c):
        return out_hbm.at[pl.ds(rbase + c * R, R)]

    pltpu.async_copy(x_slice(0), in_bufs[0], isems[0])

    def loop_body(c0, carry):
        for b in range(NBUF):
            c = c0 * NBUF + b

            @pl.when(c + 1 < CHUNKS)
            def _start_next_in():
                pltpu.async_copy(x_slice(c + 1), in_bufs[1 - b], isems[1 - b])

            pltpu.make_async_copy(x_slice(c), in_bufs[b], isems[b]).wait()

            @pl.when(c >= NBUF)
            def _wait_prev_out():
                pltpu.make_async_copy(out_bufs[b], o_slice(c), osems[b]).wait()

            @plsc.parallel_loop(0, JV, 1, unroll=2)
            def _j_body(j):
                iv = idx_v[pl.ds(j * L, L)]
                jl = j * L
                vals = [plsc.load_gather(
                            in_bufs[b],
                            [jnp.full((L,), r, jnp.int32), iv])
                        for r in range(R)]
                for r in range(R):
                    out_bufs[b][r, pl.ds(jl, L)] = vals[r]

            pltpu.async_copy(out_bufs[b], o_slice(c), osems[b])
        return carry

    lax.fori_loop(0, CHUNKS // NBUF, loop_body, 0)

    for b in range(NBUF):
        pltpu.make_async_copy(out_bufs[b], o_slice(CHUNKS - NBUF + b),
                              osems[b]).wait()


_permute = pl.kernel(
    _permute_body,
    out_type=jax.ShapeDtypeStruct((N_ROWS, N_FEAT), jnp.float32),
    mesh=plsc.VectorSubcoreMesh(
        core_axis_name="c", subcore_axis_name="s", num_cores=NC, num_subcores=NS
    ),
    scratch_types=[
        pltpu.VMEM((N_FEAT,), jnp.int32),
        pltpu.VMEM((R, N_FEAT), jnp.float32),
        pltpu.VMEM((R, N_FEAT), jnp.float32),
        pltpu.VMEM((R, N_FEAT), jnp.float32),
        pltpu.VMEM((R, N_FEAT), jnp.float32),
        pltpu.SemaphoreType.DMA,
        pltpu.SemaphoreType.DMA,
        pltpu.SemaphoreType.DMA,
        pltpu.SemaphoreType.DMA,
    ],
    compiler_params=pltpu.CompilerParams(needs_layout_passes=False),
)


@jax.jit
def kernel(x, permutation):
    perm = permutation.astype(jnp.int32)
    return _permute(x, perm)


# speedup vs baseline: 5.5050x; 1.0555x over previous
import jax
import jax.numpy as jnp
from jax import lax
from jax.experimental import pallas as pl
from jax.experimental.pallas import tpu as pltpu
from jax.experimental.pallas import tpu_sc as plsc

N_ROWS = 16384
N_FEAT = 2048
NC = 2
NS = 16
L = 16
NW = NC * NS
ROWS_PER_W = N_ROWS // NW
R = 8
CHUNKS = ROWS_PER_W // R
JV = N_FEAT // L
NIN = 4
NOUT = 2


def _permute_body(x_hbm, idx_hbm, out_hbm, idx_v, *bufs):
    in_bufs = bufs[0:NIN]
    out_bufs = bufs[NIN:NIN + NOUT]
    isems = bufs[NIN + NOUT:2 * NIN + NOUT]
    osems = bufs[2 * NIN + NOUT:2 * NIN + 2 * NOUT]

    wid = lax.axis_index("s") * NC + lax.axis_index("c")
    rbase = wid * ROWS_PER_W

    pltpu.sync_copy(idx_hbm, idx_v)

    def x_slice(c):
        return x_hbm.at[pl.ds(rbase + c * R, R)]

    def o_slice(c):
        return out_hbm.at[pl.ds(rbase + c * R, R)]

    for p in range(NIN - 1):
        pltpu.async_copy(x_slice(p), in_bufs[p], isems[p])

    def loop_body(c0, carry):
        for b in range(NIN):
            c = c0 * NIN + b
            bo = b % NOUT

            @pl.when(c + NIN - 1 < CHUNKS)
            def _start_next_in():
                nb = (b + NIN - 1) % NIN
                pltpu.async_copy(x_slice(c + NIN - 1), in_bufs[nb], isems[nb])

            pltpu.make_async_copy(x_slice(c), in_bufs[b], isems[b]).wait()

            @pl.when(c >= NOUT)
            def _wait_prev_out():
                pltpu.make_async_copy(out_bufs[bo], o_slice(c), osems[bo]).wait()

            @plsc.parallel_loop(0, JV, 1, unroll=2)
            def _j_body(j):
                iv = idx_v[pl.ds(j * L, L)]
                jl = j * L
                vals = [plsc.load_gather(
                            in_bufs[b],
                            [jnp.full((L,), r, jnp.int32), iv])
                        for r in range(R)]
                for r in range(R):
                    out_bufs[bo][r, pl.ds(jl, L)] = vals[r]

            pltpu.async_copy(out_bufs[bo], o_slice(c), osems[bo])
        return carry

    lax.fori_loop(0, CHUNKS // NIN, loop_body, 0)

    for b in range(NOUT):
        c = CHUNKS - NOUT + b
        pltpu.make_async_copy(out_bufs[c % NOUT], o_slice(c),
                              osems[c % NOUT]).wait()


_permute = pl.kernel(
    _permute_body,
    out_type=jax.ShapeDtypeStruct((N_ROWS, N_FEAT), jnp.float32),
    mesh=plsc.VectorSubcoreMesh(
        core_axis_name="c", subcore_axis_name="s", num_cores=NC, num_subcores=NS
    ),
    scratch_types=[
        pltpu.VMEM((N_FEAT,), jnp.int32),
        *[pltpu.VMEM((R, N_FEAT), jnp.float32) for _ in range(NIN + NOUT)],
        *[pltpu.SemaphoreType.DMA for _ in range(NIN + NOUT)],
    ],
    compiler_params=pltpu.CompilerParams(needs_layout_passes=False),
)


@jax.jit
def kernel(x, permutation):
    perm = permutation.astype(jnp.int32)
    return _permute(x, perm)


# speedup vs baseline: 5.5238x; 1.0034x over previous
import jax
import jax.numpy as jnp
from jax import lax
from jax.experimental import pallas as pl
from jax.experimental.pallas import tpu as pltpu
from jax.experimental.pallas import tpu_sc as plsc

N_ROWS = 16384
N_FEAT = 2048
NC = 2
NS = 16
L = 16
NW = NC * NS
ROWS_PER_W = N_ROWS // NW
R = 4
CHUNKS = ROWS_PER_W // R
JV = N_FEAT // L
NIN = 8
NOUT = 2


def _permute_body(x_hbm, idx_hbm, out_hbm, idx_v, *bufs):
    in_bufs = bufs[0:NIN]
    out_bufs = bufs[NIN:NIN + NOUT]
    isems = bufs[NIN + NOUT:2 * NIN + NOUT]
    osems = bufs[2 * NIN + NOUT:2 * NIN + 2 * NOUT]

    wid = lax.axis_index("s") * NC + lax.axis_index("c")
    rbase = wid * ROWS_PER_W

    pltpu.sync_copy(idx_hbm, idx_v)

    def x_slice(c):
        return x_hbm.at[pl.ds(rbase + c * R, R)]

    def o_slice(c):
        return out_hbm.at[pl.ds(rbase + c * R, R)]

    for p in range(NIN - 1):
        pltpu.async_copy(x_slice(p), in_bufs[p], isems[p])

    def loop_body(c0, carry):
        for b in range(NIN):
            c = c0 * NIN + b
            bo = b % NOUT

            @pl.when(c + NIN - 1 < CHUNKS)
            def _start_next_in():
                nb = (b + NIN - 1) % NIN
                pltpu.async_copy(x_slice(c + NIN - 1), in_bufs[nb], isems[nb])

            pltpu.make_async_copy(x_slice(c), in_bufs[b], isems[b]).wait()

            @pl.when(c >= NOUT)
            def _wait_prev_out():
                pltpu.make_async_copy(out_bufs[bo], o_slice(c), osems[bo]).wait()

            @plsc.parallel_loop(0, JV, 1, unroll=2)
            def _j_body(j):
                iv = idx_v[pl.ds(j * L, L)]
                jl = j * L
                vals = [plsc.load_gather(
                            in_bufs[b],
                            [jnp.full((L,), r, jnp.int32), iv])
                        for r in range(R)]
                for r in range(R):
                    out_bufs[bo][r, pl.ds(jl, L)] = vals[r]

            pltpu.async_copy(out_bufs[bo], o_slice(c), osems[bo])
        return carry

    lax.fori_loop(0, CHUNKS // NIN, loop_body, 0)

    for b in range(NOUT):
        c = CHUNKS - NOUT + b
        pltpu.make_async_copy(out_bufs[c % NOUT], o_slice(c),
                              osems[c % NOUT]).wait()


_permute = pl.kernel(
    _permute_body,
    out_type=jax.ShapeDtypeStruct((N_ROWS, N_FEAT), jnp.float32),
    mesh=plsc.VectorSubcoreMesh(
        core_axis_name="c", subcore_axis_name="s", num_cores=NC, num_subcores=NS
    ),
    scratch_types=[
        pltpu.VMEM((N_FEAT,), jnp.int32),
        *[pltpu.VMEM((R, N_FEAT), jnp.float32) for _ in range(NIN + NOUT)],
        *[pltpu.SemaphoreType.DMA for _ in range(NIN + NOUT)],
    ],
    compiler_params=pltpu.CompilerParams(needs_layout_passes=False),
)


@jax.jit
def kernel(x, permutation):
    perm = permutation.astype(jnp.int32)
    return _permute(x, perm)


# speedup vs baseline: 5.5780x; 1.0098x over previous
import jax
import jax.numpy as jnp
from jax import lax
from jax.experimental import pallas as pl
from jax.experimental.pallas import tpu as pltpu
from jax.experimental.pallas import tpu_sc as plsc

N_ROWS = 16384
N_FEAT = 2048
NC = 2
NS = 16
L = 16
NW = NC * NS
ROWS_PER_W = N_ROWS // NW
R = 4
CHUNKS = ROWS_PER_W // R
JV = N_FEAT // L
NBUF = 4


def _permute_body(x_hbm, idx_hbm, out_hbm, idx_v, *bufs):
    in_bufs = bufs[0:NBUF]
    out_bufs = bufs[NBUF:2 * NBUF]
    isems = bufs[2 * NBUF:3 * NBUF]
    osems = bufs[3 * NBUF:4 * NBUF]

    wid = lax.axis_index("s") * NC + lax.axis_index("c")
    rbase = wid * ROWS_PER_W

    pltpu.sync_copy(idx_hbm, idx_v)

    def x_slice(c):
        return x_hbm.at[pl.ds(rbase + c * R, R)]

    def o_slice(c):
        return out_hbm.at[pl.ds(rbase + c * R, R)]

    for p in range(NBUF - 1):
        pltpu.async_copy(x_slice(p), in_bufs[p], isems[p])

    def loop_body(c0, carry):
        for b in range(NBUF):
            c = c0 * NBUF + b

            @pl.when(c + NBUF - 1 < CHUNKS)
            def _start_next_in():
                nb = (b + NBUF - 1) % NBUF
                pltpu.async_copy(x_slice(c + NBUF - 1), in_bufs[nb], isems[nb])

            pltpu.make_async_copy(x_slice(c), in_bufs[b], isems[b]).wait()

            @pl.when(c >= NBUF)
            def _wait_prev_out():
                pltpu.make_async_copy(out_bufs[b], o_slice(c), osems[b]).wait()

            @plsc.parallel_loop(0, JV, 1, unroll=2)
            def _j_body(j):
                iv = idx_v[pl.ds(j * L, L)]
                jl = j * L
                vals = [plsc.load_gather(
                            in_bufs[b],
                            [jnp.full((L,), r, jnp.int32), iv])
                        for r in range(R)]
                for r in range(R):
                    out_bufs[b][r, pl.ds(jl, L)] = vals[r]

            pltpu.async_copy(out_bufs[b], o_slice(c), osems[b])
        return carry

    lax.fori_loop(0, CHUNKS // NBUF, loop_body, 0)

    for b in range(NBUF):
        pltpu.make_async_copy(out_bufs[b], o_slice(CHUNKS - NBUF + b),
                              osems[b]).wait()


_permute = pl.kernel(
    _permute_body,
    out_type=jax.ShapeDtypeStruct((N_ROWS, N_FEAT), jnp.float32),
    mesh=plsc.VectorSubcoreMesh(
        core_axis_name="c", subcore_axis_name="s", num_cores=NC, num_subcores=NS
    ),
    scratch_types=[
        pltpu.VMEM((N_FEAT,), jnp.int32),
        *[pltpu.VMEM((R, N_FEAT), jnp.float32) for _ in range(2 * NBUF)],
        *[pltpu.SemaphoreType.DMA for _ in range(2 * NBUF)],
    ],
    compiler_params=pltpu.CompilerParams(needs_layout_passes=False),
)


@jax.jit
def kernel(x, permutation):
    perm = permutation.astype(jnp.int32)
    return _permute(x, perm)
